# baseline (speedup 1.0000x reference)
import functools

import jax
import jax.numpy as jnp
from jax import lax
from jax.experimental import pallas as pl
from jax.experimental.pallas import tpu as pltpu
from jax.experimental.pallas import tpu_sc as plsc

NC = 2
NS = 16
NW = NC * NS
L = 16

N = 2097152
P = N // NW
C = 4096
NCHUNKS = P // C
CF = C * 3


def _rsqrt(x):
    i = lax.bitcast_convert_type(x, jnp.int32)
    i = jnp.int32(0x5F3759DF) - lax.shift_right_logical(i, 1)
    y = lax.bitcast_convert_type(i, jnp.float32)
    for _ in range(3):
        y = y * (1.5 - 0.5 * x * y * y)
    return y


def _sc_body(xt_hbm, out_hbm, nb_hbm, msk_hbm, in_v, out_v, nb_v, msk_v):
    c = lax.axis_index("c")
    s = lax.axis_index("s")
    wid = s * NC + c
    base_p = wid * P

    lane = lax.iota(jnp.int32, 16)
    lane3 = lane * 3

    def do_chunk(k, carry):
        off_p = base_p + k * C
        off_f = off_p * 3
        pltpu.sync_copy(xt_hbm.at[pl.ds(off_f, CF)], in_v)

        def do_group(g, carry2):
            b = g * L
            fi = lane3 + b * 3
            x = plsc.load_gather(in_v, [fi])
            y = plsc.load_gather(in_v, [fi + 1])
            z = plsc.load_gather(in_v, [fi + 2])
            r2 = x * x + y * y + z * z
            inv_r = _rsqrt(r2)
            sc = jnp.minimum(jnp.float32(1.0), 2.0 * inv_r - 1.0)
            plsc.store_scatter(out_v, [fi], x * sc)
            plsc.store_scatter(out_v, [fi + 1], y * sc)
            plsc.store_scatter(out_v, [fi + 2], z * sc)
            plsc.store_scatter(nb_v, [fi], x * inv_r)
            plsc.store_scatter(nb_v, [fi + 1], y * inv_r)
            plsc.store_scatter(nb_v, [fi + 2], z * inv_r)
            msk_v[pl.ds(b, L)] = (r2 > 1.0).astype(jnp.int32)
            return carry2

        lax.fori_loop(0, C // L, do_group, 0, unroll=4)

        pltpu.sync_copy(out_v, out_hbm.at[pl.ds(off_f, CF)])
        pltpu.sync_copy(nb_v, nb_hbm.at[pl.ds(off_f, CF)])
        pltpu.sync_copy(msk_v, msk_hbm.at[pl.ds(off_p, C)])
        return carry

    lax.fori_loop(0, NCHUNKS, do_chunk, 0)


@functools.partial(jax.jit, donate_argnums=())
def _run(xt_flat):
    mesh = plsc.VectorSubcoreMesh(core_axis_name="c", subcore_axis_name="s")
    out_flat, nb_flat, msk = pl.kernel(
        _sc_body,
        out_type=[
            jax.ShapeDtypeStruct((N * 3,), jnp.float32),
            jax.ShapeDtypeStruct((N * 3,), jnp.float32),
            jax.ShapeDtypeStruct((N,), jnp.int32),
        ],
        mesh=mesh,
        compiler_params=pltpu.CompilerParams(needs_layout_passes=False),
        scratch_types=[
            pltpu.VMEM((CF,), jnp.float32),
            pltpu.VMEM((CF,), jnp.float32),
            pltpu.VMEM((CF,), jnp.float32),
            pltpu.VMEM((C,), jnp.int32),
        ],
    )(xt_flat)
    return out_flat, nb_flat, msk


def kernel(xt):
    out_flat, nb_flat, msk = _run(xt.reshape(-1))
    out_xt = out_flat.reshape(N, 3)
    nb = nb_flat.reshape(N, 3)
    return out_xt, nb, msk.astype(bool)


# speedup vs baseline: 1.0129x; 1.0129x over previous
import functools

import jax
import jax.numpy as jnp
from jax import lax
from jax.experimental import pallas as pl
from jax.experimental.pallas import tpu as pltpu
from jax.experimental.pallas import tpu_sc as plsc

NC = 2
NS = 16
NW = NC * NS
L = 16

N = 2097152
P = N // NW
C = 4096
NCHUNKS = P // C
CF = C * 3


def _rsqrt(x):
    i = lax.bitcast_convert_type(x, jnp.int32)
    i = jnp.int32(0x5F3759DF) - lax.shift_right_logical(i, 1)
    y = lax.bitcast_convert_type(i, jnp.float32)
    for _ in range(3):
        y = y * (1.5 - 0.5 * x * y * y)
    return y


def _sc_body(xt_hbm, out_hbm, nb_hbm, msk_hbm, in_v, out_v, nb_v, msk_v):
    c = lax.axis_index("c")
    s = lax.axis_index("s")
    wid = s * NC + c
    base_p = wid * P

    lane = lax.iota(jnp.int32, 16)
    lane3 = lane * 3

    def do_chunk(k, carry):
        off_p = base_p + k * C
        off_f = off_p * 3
        pltpu.sync_copy(xt_hbm.at[pl.ds(off_f, CF)], in_v)

        @plsc.parallel_loop(0, C // L, step=1, unroll=8)
        def do_group(g):
            b = g * L
            fi = lane3 + b * 3
            x = plsc.load_gather(in_v, [fi])
            y = plsc.load_gather(in_v, [fi + 1])
            z = plsc.load_gather(in_v, [fi + 2])
            r2 = x * x + y * y + z * z
            inv_r = _rsqrt(r2)
            sc = jnp.minimum(jnp.float32(1.0), 2.0 * inv_r - 1.0)
            plsc.store_scatter(out_v, [fi], x * sc)
            plsc.store_scatter(out_v, [fi + 1], y * sc)
            plsc.store_scatter(out_v, [fi + 2], z * sc)
            plsc.store_scatter(nb_v, [fi], x * inv_r)
            plsc.store_scatter(nb_v, [fi + 1], y * inv_r)
            plsc.store_scatter(nb_v, [fi + 2], z * inv_r)
            msk_v[pl.ds(b, L)] = (r2 > 1.0).astype(jnp.int32)

        pltpu.sync_copy(out_v, out_hbm.at[pl.ds(off_f, CF)])
        pltpu.sync_copy(nb_v, nb_hbm.at[pl.ds(off_f, CF)])
        pltpu.sync_copy(msk_v, msk_hbm.at[pl.ds(off_p, C)])
        return carry

    lax.fori_loop(0, NCHUNKS, do_chunk, 0)


@functools.partial(jax.jit, donate_argnums=())
def _run(xt_flat):
    mesh = plsc.VectorSubcoreMesh(core_axis_name="c", subcore_axis_name="s")
    out_flat, nb_flat, msk = pl.kernel(
        _sc_body,
        out_type=[
            jax.ShapeDtypeStruct((N * 3,), jnp.float32),
            jax.ShapeDtypeStruct((N * 3,), jnp.float32),
            jax.ShapeDtypeStruct((N,), jnp.int32),
        ],
        mesh=mesh,
        compiler_params=pltpu.CompilerParams(needs_layout_passes=False),
        scratch_types=[
            pltpu.VMEM((CF,), jnp.float32),
            pltpu.VMEM((CF,), jnp.float32),
            pltpu.VMEM((CF,), jnp.float32),
            pltpu.VMEM((C,), jnp.int32),
        ],
    )(xt_flat)
    return out_flat, nb_flat, msk


def kernel(xt):
    out_flat, nb_flat, msk = _run(xt.reshape(-1))
    out_xt = out_flat.reshape(N, 3)
    nb = nb_flat.reshape(N, 3)
    return out_xt, nb, msk.astype(bool)


# speedup vs baseline: 1.0285x; 1.0155x over previous
import functools

import jax
import jax.numpy as jnp
from jax import lax
from jax.experimental import pallas as pl
from jax.experimental.pallas import tpu as pltpu
from jax.experimental.pallas import tpu_sc as plsc

NC = 2
NS = 16
NW = NC * NS
L = 16

N = 2097152
P = N // NW
C = 4096
NCHUNKS = P // C


def _rsqrt(x):
    i = lax.bitcast_convert_type(x, jnp.int32)
    i = jnp.int32(0x5F3759DF) - lax.shift_right_logical(i, 1)
    y = lax.bitcast_convert_type(i, jnp.float32)
    for _ in range(3):
        y = y * (1.5 - 0.5 * x * y * y)
    return y


def _sc_body(xt_hbm, out_hbm, nb_hbm, msk_hbm, in_v, out_v, nb_v, msk_v):
    c = lax.axis_index("c")
    s = lax.axis_index("s")
    wid = s * NC + c
    base_p = wid * P

    lane = lax.iota(jnp.int32, 16)
    zero = jnp.zeros((16,), jnp.int32)
    one = zero + 1
    two = zero + 2

    def do_chunk(k, carry):
        off_p = base_p + k * C
        pltpu.sync_copy(xt_hbm.at[pl.ds(off_p, C)], in_v)

        @plsc.parallel_loop(0, C // L, step=1, unroll=8)
        def do_group(g):
            pi = lane + g * L
            x = plsc.load_gather(in_v, [pi, zero])
            y = plsc.load_gather(in_v, [pi, one])
            z = plsc.load_gather(in_v, [pi, two])
            r2 = x * x + y * y + z * z
            inv_r = _rsqrt(r2)
            sc = jnp.minimum(jnp.float32(1.0), 2.0 * inv_r - 1.0)
            plsc.store_scatter(out_v, [pi, zero], x * sc)
            plsc.store_scatter(out_v, [pi, one], y * sc)
            plsc.store_scatter(out_v, [pi, two], z * sc)
            plsc.store_scatter(nb_v, [pi, zero], x * inv_r)
            plsc.store_scatter(nb_v, [pi, one], y * inv_r)
            plsc.store_scatter(nb_v, [pi, two], z * inv_r)
            msk_v[pl.ds(g * L, L)] = (r2 > 1.0).astype(jnp.int32)

        pltpu.sync_copy(out_v, out_hbm.at[pl.ds(off_p, C)])
        pltpu.sync_copy(nb_v, nb_hbm.at[pl.ds(off_p, C)])
        pltpu.sync_copy(msk_v, msk_hbm.at[pl.ds(off_p, C)])
        return carry

    lax.fori_loop(0, NCHUNKS, do_chunk, 0)


@jax.jit
def _run(xt):
    mesh = plsc.VectorSubcoreMesh(core_axis_name="c", subcore_axis_name="s")
    out_xt, nb, msk = pl.kernel(
        _sc_body,
        out_type=[
            jax.ShapeDtypeStruct((N, 3), jnp.float32),
            jax.ShapeDtypeStruct((N, 3), jnp.float32),
            jax.ShapeDtypeStruct((N,), jnp.int32),
        ],
        mesh=mesh,
        compiler_params=pltpu.CompilerParams(
            needs_layout_passes=False, use_tc_tiling_on_sc=False
        ),
        scratch_types=[
            pltpu.VMEM((C, 3), jnp.float32),
            pltpu.VMEM((C, 3), jnp.float32),
            pltpu.VMEM((C, 3), jnp.float32),
            pltpu.VMEM((C,), jnp.int32),
        ],
    )(xt)
    return out_xt, nb, msk


def kernel(xt):
    out_xt, nb, msk = _run(xt)
    return out_xt, nb, msk.astype(bool)


# speedup vs baseline: 26.3975x; 25.6651x over previous
import jax
import jax.numpy as jnp
from jax import lax
from jax.experimental import pallas as pl
from jax.experimental.pallas import tpu as pltpu
from jax.experimental.pallas import tpu_sc as plsc

NC = 2
NS = 16
NW = NC * NS
L = 16

N = 2097152
P = N // NW
C = 4096
NCHUNKS = P // C


def _rsqrt(x):
    i = lax.bitcast_convert_type(x, jnp.int32)
    i = jnp.int32(0x5F3759DF) - lax.shift_right_logical(i, 1)
    y = lax.bitcast_convert_type(i, jnp.float32)
    for _ in range(3):
        y = y * (1.5 - 0.5 * x * y * y)
    return y


def _sc_body(x_hbm, y_hbm, z_hbm,
             ox_hbm, oy_hbm, oz_hbm, nx_hbm, ny_hbm, nz_hbm, mk_hbm,
             x_v, y_v, z_v, ox_v, oy_v, oz_v, nx_v, ny_v, nz_v, mk_v):
    c = lax.axis_index("c")
    s = lax.axis_index("s")
    wid = s * NC + c
    base_p = wid * P

    def do_chunk(k, carry):
        off = base_p + k * C
        pltpu.sync_copy(x_hbm.at[pl.ds(off, C)], x_v)
        pltpu.sync_copy(y_hbm.at[pl.ds(off, C)], y_v)
        pltpu.sync_copy(z_hbm.at[pl.ds(off, C)], z_v)

        @plsc.parallel_loop(0, C // L, step=1, unroll=8)
        def do_group(g):
            b = g * L
            x = x_v[pl.ds(b, L)]
            y = y_v[pl.ds(b, L)]
            z = z_v[pl.ds(b, L)]
            r2 = x * x + y * y + z * z
            inv_r = _rsqrt(r2)
            sc = jnp.minimum(jnp.float32(1.0), 2.0 * inv_r - 1.0)
            ox_v[pl.ds(b, L)] = x * sc
            oy_v[pl.ds(b, L)] = y * sc
            oz_v[pl.ds(b, L)] = z * sc
            nx_v[pl.ds(b, L)] = x * inv_r
            ny_v[pl.ds(b, L)] = y * inv_r
            nz_v[pl.ds(b, L)] = z * inv_r
            mk_v[pl.ds(b, L)] = (r2 > 1.0).astype(jnp.int32)

        pltpu.sync_copy(ox_v, ox_hbm.at[pl.ds(off, C)])
        pltpu.sync_copy(oy_v, oy_hbm.at[pl.ds(off, C)])
        pltpu.sync_copy(oz_v, oz_hbm.at[pl.ds(off, C)])
        pltpu.sync_copy(nx_v, nx_hbm.at[pl.ds(off, C)])
        pltpu.sync_copy(ny_v, ny_hbm.at[pl.ds(off, C)])
        pltpu.sync_copy(nz_v, nz_hbm.at[pl.ds(off, C)])
        pltpu.sync_copy(mk_v, mk_hbm.at[pl.ds(off, C)])
        return carry

    lax.fori_loop(0, NCHUNKS, do_chunk, 0)


@jax.jit
def _run(x, y, z):
    mesh = plsc.VectorSubcoreMesh(core_axis_name="c", subcore_axis_name="s")
    f = jax.ShapeDtypeStruct((N,), jnp.float32)
    return pl.kernel(
        _sc_body,
        out_type=[f, f, f, f, f, f, jax.ShapeDtypeStruct((N,), jnp.int32)],
        mesh=mesh,
        compiler_params=pltpu.CompilerParams(
            needs_layout_passes=False, use_tc_tiling_on_sc=False
        ),
        scratch_types=[pltpu.VMEM((C,), jnp.float32)] * 9
        + [pltpu.VMEM((C,), jnp.int32)],
    )(x, y, z)


def kernel(xt):
    x = xt[:, 0]
    y = xt[:, 1]
    z = xt[:, 2]
    ox, oy, oz, nx, ny, nz, mk = _run(x, y, z)
    out_xt = jnp.stack([ox, oy, oz], axis=1)
    nb = jnp.stack([nx, ny, nz], axis=1)
    return out_xt, nb, mk.astype(bool)


# speedup vs baseline: 36.5585x; 1.3849x over previous
import jax
import jax.numpy as jnp
from jax import lax
from jax.experimental import pallas as pl
from jax.experimental.pallas import tpu as pltpu
from jax.experimental.pallas import tpu_sc as plsc

NC = 2
NS = 16
NW = NC * NS
L = 16

N = 2097152
P = N // NW
C = 4096
NCHUNKS = P // C


def _rsqrt(x):
    i = lax.bitcast_convert_type(x, jnp.int32)
    i = jnp.int32(0x5F3759DF) - lax.shift_right_logical(i, 1)
    y = lax.bitcast_convert_type(i, jnp.float32)
    for _ in range(3):
        y = y * (1.5 - 0.5 * x * y * y)
    return y


def _sc_body(x_hbm, y_hbm, z_hbm,
             ox_hbm, oy_hbm, oz_hbm, nx_hbm, ny_hbm, nz_hbm, mk_hbm,
             *sbuf):
    c = lax.axis_index("c")
    s = lax.axis_index("s")
    wid = s * NC + c
    base_p = wid * P

    in_hbm = (x_hbm, y_hbm, z_hbm)
    out_hbm = (ox_hbm, oy_hbm, oz_hbm, nx_hbm, ny_hbm, nz_hbm, mk_hbm)
    inb = (sbuf[0:3], sbuf[10:13])
    outb = (sbuf[3:10], sbuf[13:20])
    in_s = sbuf[20:22]
    out_s = sbuf[22:24]

    def start_in(k, b):
        off = base_p + k * C
        for h, v in zip(in_hbm, inb[b]):
            pltpu.async_copy(h.at[pl.ds(off, C)], v, in_s[b])

    def wait_in(k, b):
        off = base_p + k * C
        for h, v in zip(in_hbm, inb[b]):
            pltpu.make_async_copy(h.at[pl.ds(off, C)], v, in_s[b]).wait()

    def start_out(k, b):
        off = base_p + k * C
        for v, h in zip(outb[b], out_hbm):
            pltpu.async_copy(v, h.at[pl.ds(off, C)], out_s[b])

    def drain_out(k, b):
        off = base_p + k * C
        for v, h in zip(outb[b], out_hbm):
            pltpu.make_async_copy(v, h.at[pl.ds(off, C)], out_s[b]).wait()

    start_in(0, 0)

    def do_pair(k2, carry):
        for b in range(2):
            k = k2 * 2 + b

            @pl.when(k + 1 < NCHUNKS)
            def _():
                start_in(k + 1, 1 - b)

            wait_in(k, b)

            @pl.when(k >= 2)
            def _():
                drain_out(k - 2, b)

            x_v, y_v, z_v = inb[b]
            ox_v, oy_v, oz_v, nx_v, ny_v, nz_v, mk_v = outb[b]

            @plsc.parallel_loop(0, C // L, step=1, unroll=8)
            def do_group(g):
                gb = g * L
                x = x_v[pl.ds(gb, L)]
                y = y_v[pl.ds(gb, L)]
                z = z_v[pl.ds(gb, L)]
                r2 = x * x + y * y + z * z
                inv_r = _rsqrt(r2)
                sc = jnp.minimum(jnp.float32(1.0), 2.0 * inv_r - 1.0)
                ox_v[pl.ds(gb, L)] = x * sc
                oy_v[pl.ds(gb, L)] = y * sc
                oz_v[pl.ds(gb, L)] = z * sc
                nx_v[pl.ds(gb, L)] = x * inv_r
                ny_v[pl.ds(gb, L)] = y * inv_r
                nz_v[pl.ds(gb, L)] = z * inv_r
                mk_v[pl.ds(gb, L)] = (r2 > 1.0).astype(jnp.int32)

            start_out(k, b)
        return carry

    lax.fori_loop(0, NCHUNKS // 2, do_pair, 0)
    drain_out(NCHUNKS - 2, 0)
    drain_out(NCHUNKS - 1, 1)


@jax.jit
def _run(x, y, z):
    mesh = plsc.VectorSubcoreMesh(core_axis_name="c", subcore_axis_name="s")
    f = jax.ShapeDtypeStruct((N,), jnp.float32)
    return pl.kernel(
        _sc_body,
        out_type=[f, f, f, f, f, f, jax.ShapeDtypeStruct((N,), jnp.int32)],
        mesh=mesh,
        compiler_params=pltpu.CompilerParams(
            needs_layout_passes=False, use_tc_tiling_on_sc=False
        ),
        scratch_types=(
            [pltpu.VMEM((C,), jnp.float32)] * 9
            + [pltpu.VMEM((C,), jnp.int32)]
        ) * 2
        + [pltpu.SemaphoreType.DMA] * 4,
    )(x, y, z)


def kernel(xt):
    x = xt[:, 0]
    y = xt[:, 1]
    z = xt[:, 2]
    ox, oy, oz, nx, ny, nz, mk = _run(x, y, z)
    out_xt = jnp.stack([ox, oy, oz], axis=1)
    nb = jnp.stack([nx, ny, nz], axis=1)
    return out_xt, nb, mk.astype(bool)
